# baseline (speedup 1.0000x reference)
import functools

import jax
import jax.numpy as jnp
from jax import lax
from jax.experimental import pallas as pl
from jax.experimental.pallas import tpu as pltpu
from jax.experimental.pallas import tpu_sc as plsc

B = 4096
L = 200
D = 128
NC = 2
NS = 16
NW = NC * NS
BPW = B // NW
CH = 40
NG = L // CH
NCH = D // 16


def _build():
  mesh = plsc.VectorSubcoreMesh(core_axis_name="c", subcore_axis_name="s")

  @functools.partial(
      pl.kernel,
      out_type=jax.ShapeDtypeStruct((B, D), jnp.float32),
      mesh=mesh,
      scratch_types=[
          pltpu.VMEM((BPW * L,), jnp.int32),
          pltpu.VMEM((2, CH, D), jnp.float32),
          pltpu.VMEM((BPW, D), jnp.float32),
          pltpu.SemaphoreType.DMA,
          pltpu.SemaphoreType.DMA,
      ],
  )
  def k(table_hbm, idx_hbm, out_hbm, idx_v, rows_v, out_v, sem0, sem1):
    wid = lax.axis_index("c") * NS + lax.axis_index("s")
    base = wid * BPW
    pltpu.sync_copy(idx_hbm.at[pl.ds(base * L, BPW * L)], idx_v)
    sems = (sem0, sem1)

    def zero(buf):
      zv = jnp.zeros((16,), jnp.float32)

      @pl.loop(0, CH)
      def _(r):
        for c in range(NCH):
          rows_v[buf, r, pl.ds(c * 16, 16)] = zv

    def start(bb, buf):
      off = pl.multiple_of(bb * L, 8)
      for g in range(NG):
        pltpu.async_copy(table_hbm.at[idx_v.at[pl.ds(off + g * CH, CH)]],
                         rows_v.at[buf], sems[buf], add=True)

    def wait(bb, buf):
      off = pl.multiple_of(bb * L, 8)
      for g in range(NG):
        pltpu.make_async_copy(table_hbm.at[idx_v.at[pl.ds(off + g * CH, CH)]],
                              rows_v.at[buf], sems[buf]).wait()

    zero(0)
    zero(1)
    start(0, 0)
    start(1, 1)

    @pl.loop(0, BPW, step=2)
    def _pair(b):
      for ph in range(2):
        bb = b + ph
        wait(bb, ph)
        r1 = rows_v.at[ph]

        def add1(r, accs):
          return tuple(accs[c] + r1[r, pl.ds(c * 16, 16)]
                       for c in range(NCH))

        accs = tuple(r1[0, pl.ds(c * 16, 16)] for c in range(NCH))
        accs = lax.fori_loop(1, CH, add1, accs, unroll=4)
        scale = jnp.float32(1.0 / L)
        for c in range(NCH):
          out_v[bb, pl.ds(c * 16, 16)] = accs[c] * scale

        zero(ph)

        @pl.when(bb + 2 < BPW)
        def _():
          start(bb + 2, ph)

    pltpu.sync_copy(out_v, out_hbm.at[pl.ds(base, BPW)])

  return k


def kernel(sentences, offsets, weight):
  del offsets
  idx_flat = sentences.reshape(-1)
  return _build()(weight, idx_flat)


# speedup vs baseline: 1.2086x; 1.2086x over previous
import functools

import jax
import jax.numpy as jnp
from jax import lax
from jax.experimental import pallas as pl
from jax.experimental.pallas import tpu as pltpu
from jax.experimental.pallas import tpu_sc as plsc

B = 4096
L = 200
D = 128
NC = 2
NS = 16
NW = NC * NS
BPW = B // NW
CH = 40
NG = L // CH
NCH = D // 16
NBUF = 4


def _build():
  mesh = plsc.VectorSubcoreMesh(core_axis_name="c", subcore_axis_name="s")

  @functools.partial(
      pl.kernel,
      out_type=jax.ShapeDtypeStruct((B, D), jnp.float32),
      mesh=mesh,
      scratch_types=[
          pltpu.VMEM((BPW * L,), jnp.int32),
          pltpu.VMEM((NBUF, CH, D), jnp.float32),
          pltpu.VMEM((BPW, D), jnp.float32),
      ] + [pltpu.SemaphoreType.DMA] * NBUF,
  )
  def k(table_hbm, idx_hbm, out_hbm, idx_v, rows_v, out_v, *sems):
    wid = lax.axis_index("c") * NS + lax.axis_index("s")
    base = wid * BPW
    pltpu.sync_copy(idx_hbm.at[pl.ds(base * L, BPW * L)], idx_v)

    def zero(buf):
      zv = jnp.zeros((16,), jnp.float32)

      @pl.loop(0, CH)
      def _(r):
        for c in range(NCH):
          rows_v[buf, r, pl.ds(c * 16, 16)] = zv

    def start(bb, buf):
      off = pl.multiple_of(bb * L, 8)
      for g in range(NG):
        pltpu.async_copy(table_hbm.at[idx_v.at[pl.ds(off + g * CH, CH)]],
                         rows_v.at[buf], sems[buf], add=True)

    def wait(bb, buf):
      off = pl.multiple_of(bb * L, 8)
      for g in range(NG):
        pltpu.make_async_copy(table_hbm.at[idx_v.at[pl.ds(off + g * CH, CH)]],
                              rows_v.at[buf], sems[buf]).wait()

    for buf in range(NBUF):
      zero(buf)
      start(buf, buf)

    @pl.loop(0, BPW, step=NBUF)
    def _pair(b):
      for ph in range(NBUF):
        bb = b + ph
        wait(bb, ph)
        r1 = rows_v.at[ph]

        def add1(r, accs):
          return tuple(accs[c] + r1[r, pl.ds(c * 16, 16)]
                       for c in range(NCH))

        accs = tuple(r1[0, pl.ds(c * 16, 16)] for c in range(NCH))
        accs = lax.fori_loop(1, CH, add1, accs, unroll=4)
        scale = jnp.float32(1.0 / L)
        for c in range(NCH):
          out_v[bb, pl.ds(c * 16, 16)] = accs[c] * scale

        zero(ph)

        @pl.when(bb + NBUF < BPW)
        def _():
          start(bb + NBUF, ph)

    pltpu.sync_copy(out_v, out_hbm.at[pl.ds(base, BPW)])

  return k


def kernel(sentences, offsets, weight):
  del offsets
  idx_flat = sentences.reshape(-1)
  return _build()(weight, idx_flat)


# speedup vs baseline: 1.2577x; 1.0406x over previous
import functools

import jax
import jax.numpy as jnp
from jax import lax
from jax.experimental import pallas as pl
from jax.experimental.pallas import tpu as pltpu
from jax.experimental.pallas import tpu_sc as plsc

B = 4096
L = 200
D = 128
NC = 2
NS = 16
NW = NC * NS
BPW = B // NW
CH = 40
NG = L // CH
NCH = D // 16
NBUF = 8


def _build():
  mesh = plsc.VectorSubcoreMesh(core_axis_name="c", subcore_axis_name="s")

  @functools.partial(
      pl.kernel,
      out_type=jax.ShapeDtypeStruct((B, D), jnp.float32),
      mesh=mesh,
      scratch_types=[
          pltpu.VMEM((BPW * L,), jnp.int32),
          pltpu.VMEM((NBUF, CH, D), jnp.float32),
          pltpu.VMEM((BPW, D), jnp.float32),
      ] + [pltpu.SemaphoreType.DMA] * NBUF,
  )
  def k(table_hbm, idx_hbm, out_hbm, idx_v, rows_v, out_v, *sems):
    wid = lax.axis_index("c") * NS + lax.axis_index("s")
    base = wid * BPW
    pltpu.sync_copy(idx_hbm.at[pl.ds(base * L, BPW * L)], idx_v)

    def zero(buf):
      zv = jnp.zeros((16,), jnp.float32)

      @pl.loop(0, CH)
      def _(r):
        for c in range(NCH):
          rows_v[buf, r, pl.ds(c * 16, 16)] = zv

    def start(bb, buf):
      off = pl.multiple_of(bb * L, 8)
      for g in range(NG):
        pltpu.async_copy(table_hbm.at[idx_v.at[pl.ds(off + g * CH, CH)]],
                         rows_v.at[buf], sems[buf], add=True)

    def wait(bb, buf):
      off = pl.multiple_of(bb * L, 8)
      for g in range(NG):
        pltpu.make_async_copy(table_hbm.at[idx_v.at[pl.ds(off + g * CH, CH)]],
                              rows_v.at[buf], sems[buf]).wait()

    for buf in range(NBUF):
      zero(buf)
      start(buf, buf)

    @pl.loop(0, BPW, step=NBUF)
    def _pair(b):
      for ph in range(NBUF):
        bb = b + ph
        wait(bb, ph)
        r1 = rows_v.at[ph]

        def add1(r, accs):
          return tuple(accs[c] + r1[r, pl.ds(c * 16, 16)]
                       for c in range(NCH))

        accs = tuple(r1[0, pl.ds(c * 16, 16)] for c in range(NCH))
        accs = lax.fori_loop(1, CH, add1, accs, unroll=4)
        scale = jnp.float32(1.0 / L)
        for c in range(NCH):
          out_v[bb, pl.ds(c * 16, 16)] = accs[c] * scale

        zero(ph)

        @pl.when(bb + NBUF < BPW)
        def _():
          start(bb + NBUF, ph)

    pltpu.sync_copy(out_v, out_hbm.at[pl.ds(base, BPW)])

  return k


def kernel(sentences, offsets, weight):
  del offsets
  idx_flat = sentences.reshape(-1)
  return _build()(weight, idx_flat)


# speedup vs baseline: 1.2806x; 1.0182x over previous
import functools

import jax
import jax.numpy as jnp
from jax import lax
from jax.experimental import pallas as pl
from jax.experimental.pallas import tpu as pltpu
from jax.experimental.pallas import tpu_sc as plsc

B = 4096
L = 200
D = 128
NC = 2
NS = 16
NW = NC * NS
BPW = B // NW
CHUNKS = ((0, 104), (104, 96))
CH = CHUNKS[0][1]
NCH = D // 16
NBUF = 4


def _build():
  mesh = plsc.VectorSubcoreMesh(core_axis_name="c", subcore_axis_name="s")

  @functools.partial(
      pl.kernel,
      out_type=jax.ShapeDtypeStruct((B, D), jnp.float32),
      mesh=mesh,
      scratch_types=[
          pltpu.VMEM((BPW * L,), jnp.int32),
          pltpu.VMEM((NBUF, CH, D), jnp.float32),
          pltpu.VMEM((BPW, D), jnp.float32),
      ] + [pltpu.SemaphoreType.DMA] * NBUF,
  )
  def k(table_hbm, idx_hbm, out_hbm, idx_v, rows_v, out_v, *sems):
    wid = lax.axis_index("c") * NS + lax.axis_index("s")
    base = wid * BPW
    pltpu.sync_copy(idx_hbm.at[pl.ds(base * L, BPW * L)], idx_v)

    def zero(buf):
      zv = jnp.zeros((16,), jnp.float32)

      @pl.loop(0, CH)
      def _(r):
        for c in range(NCH):
          rows_v[buf, r, pl.ds(c * 16, 16)] = zv

    def start(bb, buf):
      off = pl.multiple_of(bb * L, 8)
      for g, n in CHUNKS:
        pltpu.async_copy(table_hbm.at[idx_v.at[pl.ds(off + g, n)]],
                         rows_v.at[buf].at[pl.ds(0, n)], sems[buf], add=True)

    def wait(bb, buf):
      off = pl.multiple_of(bb * L, 8)
      for g, n in CHUNKS:
        pltpu.make_async_copy(table_hbm.at[idx_v.at[pl.ds(off + g, n)]],
                              rows_v.at[buf].at[pl.ds(0, n)],
                              sems[buf]).wait()

    for buf in range(NBUF):
      zero(buf)
      start(buf, buf)

    @pl.loop(0, BPW, step=NBUF)
    def _pair(b):
      for ph in range(NBUF):
        bb = b + ph
        wait(bb, ph)
        r1 = rows_v.at[ph]

        def add1(r, accs):
          return tuple(accs[c] + r1[r, pl.ds(c * 16, 16)]
                       for c in range(NCH))

        accs = tuple(r1[0, pl.ds(c * 16, 16)] for c in range(NCH))
        accs = lax.fori_loop(1, CH, add1, accs, unroll=4)
        scale = jnp.float32(1.0 / L)
        for c in range(NCH):
          out_v[bb, pl.ds(c * 16, 16)] = accs[c] * scale

        zero(ph)

        @pl.when(bb + NBUF < BPW)
        def _():
          start(bb + NBUF, ph)

    pltpu.sync_copy(out_v, out_hbm.at[pl.ds(base, BPW)])

  return k


def kernel(sentences, offsets, weight):
  del offsets
  idx_flat = sentences.reshape(-1)
  return _build()(weight, idx_flat)
